# baseline (speedup 1.0000x reference)
import functools

import jax
import jax.numpy as jnp
from jax import lax
from jax.experimental import pallas as pl
from jax.experimental.pallas import tpu as pltpu
from jax.experimental.pallas import tpu_sc as plsc

B = 4096
F = 26
D = 16
V = 100000
NUM = 13
H1, H2 = 512, 256

NC, NS = 2, 16
NW = NC * NS
BF = B * F
PW = BF // NW
CH = 128
NCH = PW // CH


def _sc_gather_body(idx_hbm, idx16_hbm, emb_hbm, lr16_hbm, out_emb, out_lr,
                    idx_v, idx16_v, rows_v, lrows_v, sem):
    wid = lax.axis_index("s") * NC + lax.axis_index("c")
    pltpu.sync_copy(idx_hbm.at[wid], idx_v)
    pltpu.sync_copy(idx16_hbm.at[wid], idx16_v)

    def fire(j, carry):
        pltpu.async_copy(emb_hbm.at[idx_v.at[j]],
                         rows_v.at[pl.ds(j * CH, CH)], sem)
        pltpu.async_copy(lr16_hbm.at[idx16_v.at[j]],
                         lrows_v.at[pl.ds(j * CH, CH)], sem)
        return carry

    lax.fori_loop(0, NCH, fire, 0)
    pltpu.make_async_copy(emb_hbm.at[pl.ds(0, PW)], rows_v, sem).wait()
    pltpu.make_async_copy(lr16_hbm.at[pl.ds(0, PW)], lrows_v, sem).wait()
    pltpu.sync_copy(rows_v, out_emb.at[wid])
    pltpu.sync_copy(lrows_v, out_lr.at[wid])


@functools.lru_cache(maxsize=None)
def _sc_gather():
    mesh = plsc.VectorSubcoreMesh(core_axis_name="c", subcore_axis_name="s")
    return pl.kernel(
        _sc_gather_body,
        mesh=mesh,
        compiler_params=pltpu.CompilerParams(use_tc_tiling_on_sc=False),
        out_type=(
            jax.ShapeDtypeStruct((NW, PW, D), jnp.float32),
            jax.ShapeDtypeStruct((NW, PW, D), jnp.float32),
        ),
        scratch_types=[
            pltpu.VMEM((NCH, CH), jnp.int32),
            pltpu.VMEM((NCH, CH), jnp.int32),
            pltpu.VMEM((PW, D), jnp.float32),
            pltpu.VMEM((PW, D), jnp.float32),
            pltpu.SemaphoreType.DMA,
        ],
    )


TB = 512


def _dotT(x, w):
    return lax.dot_general(x, w, (((1,), (1,)), ((), ())),
                           precision=lax.Precision.HIGHEST,
                           preferred_element_type=jnp.float32)


def _dot(x, w):
    return lax.dot_general(x, w, (((1,), (0,)), ((), ())),
                           precision=lax.Precision.HIGHEST,
                           preferred_element_type=jnp.float32)


def _tc_body(flat_ref, xnum_ref, lr16_ref, lane_ref, w1e_ref, w1n_ref, b1_ref,
             w2_ref, b2_ref, w3_ref, b3_ref, lrw_ref, lrb_ref, out_ref):
    flat = flat_ref[...]
    xnum = xnum_ref[...]
    h = _dotT(flat, w1e_ref[...]) + _dotT(xnum, w1n_ref[...]) + b1_ref[...]
    h = jnp.maximum(h, 0.0)
    h = jnp.maximum(_dotT(h, w2_ref[...]) + b2_ref[...], 0.0)
    dnn = jnp.sum(h * w3_ref[...], axis=1, keepdims=True) + b3_ref[0, 0]
    jj = lax.broadcasted_iota(jnp.int32, (F * D, D), 0)
    dd = lax.broadcasted_iota(jnp.int32, (F * D, D), 1)
    p = jnp.where(jj % D == dd, 1.0, 0.0).astype(jnp.float32)
    sum_emb = _dot(flat, p)
    sum_sq = jnp.sum(sum_emb * sum_emb, axis=1, keepdims=True)
    sq_sum = jnp.sum(flat * flat, axis=1, keepdims=True)
    fm = 0.5 * (sum_sq - sq_sum)
    ff = lax.broadcasted_iota(jnp.int32, (F, F * D), 0)
    cc = lax.broadcasted_iota(jnp.int32, (F, F * D), 1)
    e = jnp.where(cc // D == ff, 1.0, 0.0).astype(jnp.float32)
    lane_exp = _dot(lane_ref[...], e)
    mod16 = (lax.broadcasted_iota(jnp.int32, (TB, F * D), 1) % D
             ).astype(jnp.float32)
    sel = jnp.where(lane_exp == mod16, 1.0, 0.0).astype(jnp.float32)
    lr_sum = jnp.sum(lr16_ref[...] * sel, axis=1, keepdims=True)
    lin = (lrb_ref[0, 0] + lr_sum
           + jnp.sum(xnum * lrw_ref[...], axis=1, keepdims=True))
    out_ref[...] = dnn + fm + lin


@functools.lru_cache(maxsize=None)
def _tc_call():
    grid = (B // TB,)
    row = lambda i: (i, 0)
    rep = lambda i: (0, 0)
    return pl.pallas_call(
        _tc_body,
        grid=grid,
        in_specs=[
            pl.BlockSpec((TB, F * D), row),
            pl.BlockSpec((TB, NUM), row),
            pl.BlockSpec((TB, F * D), row),
            pl.BlockSpec((TB, F), row),
            pl.BlockSpec((H1, F * D), rep),
            pl.BlockSpec((H1, NUM), rep),
            pl.BlockSpec((1, H1), rep),
            pl.BlockSpec((H2, H1), rep),
            pl.BlockSpec((1, H2), rep),
            pl.BlockSpec((1, H2), rep),
            pl.BlockSpec((1, 1), rep),
            pl.BlockSpec((1, NUM), rep),
            pl.BlockSpec((1, 1), rep),
        ],
        out_specs=pl.BlockSpec((TB, 1), row),
        out_shape=jax.ShapeDtypeStruct((B, 1), jnp.float32),
    )


def kernel(x_cat, x_num, emb_tables, lr_tables, lr_w, lr_bias,
           W1, b1, W2, b2, W3, b3):
    offs = (jnp.arange(F, dtype=jnp.int32) * V)[None, :]
    idx = x_cat.astype(jnp.int32) + offs
    emb_flat = emb_tables.reshape(F * V, D)
    lr16_flat = lr_tables.reshape(F * V // D, D)
    lane_f = (idx % D).astype(jnp.float32)
    ge, gl = _sc_gather()(idx.reshape(NW, NCH, CH),
                          (idx // D).reshape(NW, NCH, CH),
                          emb_flat, lr16_flat)
    flat = ge.reshape(B, F * D)
    lr16 = gl.reshape(B, F * D)
    return _tc_call()(
        flat, x_num, lr16, lane_f,
        W1[:, :F * D], W1[:, F * D:], b1.reshape(1, H1),
        W2, b2.reshape(1, H2),
        W3, b3.reshape(1, 1),
        lr_w, lr_bias.reshape(1, 1),
    )
